# baseline (speedup 1.0000x reference)
import functools

import jax
import jax.numpy as jnp
from jax.experimental import pallas as pl
from jax.experimental.pallas import tpu as pltpu
from jax.experimental.pallas import tpu_sc as plsc

H = W = 180
C = 256
KDIM = 9 * C
NCLS = 10
NCP = 16
K = 500
HW = H * W
HWP = 32768
BM = 1080
NB = HW // BM
NBN = 15
BN = HW // NBN
KPAD = 512
SC_NC, SC_NS = 2, 16
SC_BPW = KPAD // (SC_NC * SC_NS)

SHIFTS = (-W - 1, -W, -W + 1, -1, 1, W - 1, W, W + 1)


def _conv1_kernel(a_ref, w_ref, b_ref, out_ref):
    a = a_ref[...].astype(jnp.bfloat16)
    w = w_ref[...].astype(jnp.bfloat16)
    acc = jnp.dot(a, w, preferred_element_type=jnp.float32)
    out_ref[...] = jnp.maximum(acc + b_ref[...], 0.0)


def _conv2_kernel(a_ref, w_ref, b_ref, out_ref):
    a = a_ref[...].astype(jnp.bfloat16)
    w = w_ref[...].astype(jnp.bfloat16)
    acc = jnp.dot(a, w, preferred_element_type=jnp.float32)
    out_ref[...] = jax.nn.sigmoid(acc + b_ref[...])


def _nms_kernel(c_ref, n0, n1, n2, n3, n4, n5, n6, n7, out_ref):
    i = pl.program_id(0)
    hm = c_ref[...]
    m_iota = (jax.lax.broadcasted_iota(jnp.int32, (BN, 1), 0) + i * BN)
    x_pos = jax.lax.rem(m_iota, W)
    nrefs = (n0, n1, n2, n3, n4, n5, n6, n7)
    dxs = (-1, 0, 1, -1, 1, -1, 0, 1)
    hmax = hm
    for j in range(8):
        nb = nrefs[j][...]
        dx = dxs[j]
        if dx == 1:
            nb = jnp.where(x_pos == W - 1, -1.0, nb)
        elif dx == -1:
            nb = jnp.where(x_pos == 0, -1.0, nb)
        hmax = jnp.maximum(hmax, nb)
    lane = jax.lax.broadcasted_iota(jnp.int32, (BN, NCP), 1)
    z = jnp.where(hm >= hmax, hm, 0.0)
    out_ref[...] = jnp.where(lane < NCLS, z, -1.0)


def _topk_kernel(zp_ref, scores_ref, xs_ref, ys_ref, cls_ref, pos_ref,
                 zf_ref, bm_ref):
    zf_ref[...] = zp_ref[...]
    bm_ref[...] = jnp.max(zp_ref[...].reshape(256, 128, NCP), axis=2)

    bidx = (jax.lax.broadcasted_iota(jnp.int32, (256, 128), 0) * 128
            + jax.lax.broadcasted_iota(jnp.int32, (256, 128), 1))
    lane128 = jax.lax.broadcasted_iota(jnp.int32, (1, 128), 1)
    lane16 = jax.lax.broadcasted_iota(jnp.int32, (1, NCP), 1)
    big = jnp.int32(10 ** 9)

    def full_scan(v):
        zf = zf_ref[...]
        cm = (jax.lax.broadcasted_iota(jnp.int32, (HWP, NCP), 0)
              + jax.lax.broadcasted_iota(jnp.int32, (HWP, NCP), 1) * HW)
        f = jnp.min(jnp.where(zf >= v, cm, big))
        c = f // HW
        return c, f - c * HW

    def fast(args):
        v, m1 = args
        row = zf_ref[pl.ds(m1, 1), :]
        c1 = jnp.min(jnp.where(row >= v, lane16, big))
        return c1, m1

    def body(s, carry):
        bm = bm_ref[...]
        v = jnp.max(bm)
        hits = (bm >= v)
        nb = jnp.sum(hits.astype(jnp.int32))
        m1 = jnp.min(jnp.where(hits, bidx, big))
        c, m = jax.lax.cond(nb > 1, lambda a: full_scan(a[0]), fast, (v, m1))
        row = zf_ref[pl.ds(m, 1), :]
        rnew = jnp.where(lane16 == c, -1.0, row)
        zf_ref[pl.ds(m, 1), :] = rnew
        ms = m // 128
        ml = m - ms * 128
        bmrow = bm_ref[pl.ds(ms, 1), :]
        bm_ref[pl.ds(ms, 1), :] = jnp.where(lane128 == ml,
                                            jnp.max(rnew), bmrow)
        y = m // W
        x = m - y * W
        scores_ref[0, s] = v
        xs_ref[0, s] = x.astype(jnp.float32)
        ys_ref[0, s] = y.astype(jnp.float32)
        cls_ref[0, s] = c
        pos_ref[0, s] = m
        return carry

    jax.lax.fori_loop(0, K, body, None)


def _sc_gather_kernel(table_hbm, idx_hbm, out_hbm, idx_v, rows_v, sem):
    wid = jax.lax.axis_index("s") * SC_NC + jax.lax.axis_index("c")
    base = wid * SC_BPW
    pltpu.sync_copy(idx_hbm.at[pl.ds(base, SC_BPW)], idx_v)
    pltpu.async_copy(table_hbm.at[idx_v], rows_v, sem).wait()
    pltpu.sync_copy(rows_v, out_hbm.at[pl.ds(base, SC_BPW)])


def _im2col(arr_hwc):
    views = []
    for t in range(9):
        dy, dx = t // 3, t % 3
        views.append(arr_hwc[dy:dy + H, dx:dx + W, :].reshape(HW, -1))
    return jnp.concatenate(views, axis=1)


def _conv_call(body, a2, w2, b2, ncols):
    return pl.pallas_call(
        body,
        grid=(NB,),
        in_specs=[
            pl.BlockSpec((BM, KDIM), lambda i: (i, 0)),
            pl.BlockSpec((KDIM, ncols), lambda i: (0, 0)),
            pl.BlockSpec((1, ncols), lambda i: (0, 0)),
        ],
        out_specs=pl.BlockSpec((BM, ncols), lambda i: (i, 0)),
        out_shape=jax.ShapeDtypeStruct((HW, ncols), jnp.float32),
    )(a2, w2, b2)


@jax.jit
def kernel(x, W_shared, b_shared, W_hm, b_hm):
    xt = jnp.transpose(x[0], (1, 2, 0))
    xp = jnp.pad(xt, ((1, 1), (1, 1), (0, 0)))
    a2 = _im2col(xp)
    w2 = jnp.transpose(W_shared, (2, 3, 1, 0)).reshape(KDIM, C)
    feat2d = _conv_call(_conv1_kernel, a2, w2, b_shared[None, :], C)

    fp = jnp.pad(feat2d.reshape(H, W, C), ((1, 1), (1, 1), (0, 0)))
    a2h = _im2col(fp)
    w2h = jnp.transpose(W_hm, (2, 3, 1, 0)).reshape(KDIM, NCLS)
    w2h = jnp.pad(w2h, ((0, 0), (0, NCP - NCLS)))
    b16 = jnp.concatenate([b_hm, jnp.full((NCP - NCLS,), -30.0,
                                          jnp.float32)])[None, :]
    hm2d = _conv_call(_conv2_kernel, a2h, w2h, b16, NCP)

    pad = jnp.full((W + 1, NCP), -1.0, jnp.float32)
    hmp = jnp.concatenate([pad, hm2d, pad], axis=0)
    nviews = [jax.lax.dynamic_slice(hmp, (W + 1 + d, 0), (HW, NCP))
              for d in SHIFTS]
    nblk = pl.BlockSpec((BN, NCP), lambda i: (i, 0))
    zmap = pl.pallas_call(
        _nms_kernel,
        grid=(NBN,),
        in_specs=[nblk] * 9,
        out_specs=nblk,
        out_shape=jax.ShapeDtypeStruct((HW, NCP), jnp.float32),
    )(hm2d, *nviews)

    zp = jnp.concatenate(
        [zmap, jnp.full((HWP - HW, NCP), -1.0, jnp.float32)], axis=0)
    scores, xs, ys, clses, pos = pl.pallas_call(
        _topk_kernel,
        in_specs=[pl.BlockSpec((HWP, NCP), lambda: (0, 0))],
        out_specs=[
            pl.BlockSpec(memory_space=pltpu.SMEM),
            pl.BlockSpec(memory_space=pltpu.SMEM),
            pl.BlockSpec(memory_space=pltpu.SMEM),
            pl.BlockSpec(memory_space=pltpu.SMEM),
            pl.BlockSpec(memory_space=pltpu.SMEM),
        ],
        out_shape=[
            jax.ShapeDtypeStruct((1, K), jnp.float32),
            jax.ShapeDtypeStruct((1, K), jnp.float32),
            jax.ShapeDtypeStruct((1, K), jnp.float32),
            jax.ShapeDtypeStruct((1, K), jnp.int32),
            jax.ShapeDtypeStruct((1, K), jnp.int32),
        ],
        scratch_shapes=[
            pltpu.VMEM((HWP, NCP), jnp.float32),
            pltpu.VMEM((256, 128), jnp.float32),
        ],
    )(zp)

    posp = jnp.concatenate(
        [pos[0], jnp.zeros((KPAD - K,), jnp.int32)])
    ct = _gather(feat2d, posp)

    return ct[None, :K], scores, xs, ys, clses


def _gather(feat2d, posp):
    mesh = plsc.VectorSubcoreMesh(core_axis_name="c", subcore_axis_name="s")
    sc_gather = functools.partial(
        pl.kernel, mesh=mesh,
        out_type=jax.ShapeDtypeStruct((KPAD, C), jnp.float32),
        scratch_types=[
            pltpu.VMEM((SC_BPW,), jnp.int32),
            pltpu.VMEM((SC_BPW, C), jnp.float32),
            pltpu.SemaphoreType.DMA,
        ],
    )(_sc_gather_kernel)
    return sc_gather(feat2d, posp)
